# baseline (speedup 1.0000x reference)
import functools

import jax
import jax.numpy as jnp
from jax import lax
from jax.experimental import pallas as pl
from jax.experimental.pallas import tpu as pltpu
from jax.experimental.pallas import tpu_sc as plsc

VOCAB = 1000000
EMBED = 64
BATCH = 16384

NUM_CORES = 2
NUM_SUBCORES = 16
NUM_WORKERS = NUM_CORES * NUM_SUBCORES
B_PER_WORKER = BATCH // NUM_WORKERS
LANES = 128
RING = 8
STEP = 16
N_STEPS = B_PER_WORKER // STEP

_mesh = plsc.VectorSubcoreMesh(core_axis_name="c", subcore_axis_name="s")


@functools.partial(
    pl.kernel,
    mesh=_mesh,
    out_type=jax.ShapeDtypeStruct((EMBED, BATCH), jnp.float32),
    scratch_types=[
        pltpu.VMEM((B_PER_WORKER,), jnp.int32),
        pltpu.VMEM((RING, EMBED, LANES), jnp.float32),
        pltpu.VMEM((EMBED, B_PER_WORKER), jnp.float32),
    ]
    + [pltpu.SemaphoreType.DMA] * RING,
    compiler_params=pltpu.CompilerParams(
        use_tc_tiling_on_sc=True, needs_layout_passes=False
    ),
)
def _gather_kernel(idx_hbm, tableT_hbm, outT_hbm, idx_v, blocks, colsT_v, *sems):
    wid = lax.axis_index("s") * NUM_CORES + lax.axis_index("c")
    base = wid * B_PER_WORKER
    pltpu.sync_copy(idx_hbm.at[pl.ds(base, B_PER_WORKER)], idx_v)

    row_vecs = [lax.iota(jnp.int32, 16) + (16 * j) for j in range(EMBED // 16)]

    def fire(b, r):
        blk = pl.multiple_of(lax.div(r, LANES) * 0, LANES)
        pltpu.async_copy(
            tableT_hbm.at[:, pl.ds(blk, LANES)],
            blocks.at[b],
            sems[b],
        )

    def wait_extract(b, r, pos):
        pltpu.make_async_copy(
            tableT_hbm.at[:, pl.ds(0, LANES)],
            blocks.at[b],
            sems[b],
        ).wait()
        lane = jnp.bitwise_and(r, LANES - 1)
        col_ids = jnp.full((16,), lane, jnp.int32)
        pos_ids = jnp.full((16,), pos, jnp.int32)
        del col_ids, pos_ids

    def step(s, vec_old):
        vec_cur = idx_v[pl.ds(s * STEP, STEP)]
        for k in range(STEP):
            b = k % RING
            if k < RING:
                @pl.when(s > 0)
                def _(k=k, b=b):
                    wait_extract(b, vec_old[k + RING], (s - 1) * STEP + k + RING)
            else:
                wait_extract(b, vec_cur[k - RING], s * STEP + k - RING)
            fire(b, vec_cur[k])
        return vec_cur

    lax.fori_loop(0, N_STEPS, step, idx_v[pl.ds(0, STEP)])

    vec_tail = idx_v[pl.ds(B_PER_WORKER - STEP, STEP)]
    for k in range(RING):
        wait_extract(k, vec_tail[k + RING], B_PER_WORKER - RING + k)

    pltpu.sync_copy(colsT_v, outT_hbm.at[:, pl.ds(base, B_PER_WORKER)])


def kernel(inputs, embeddings):
    idx = inputs.astype(jnp.int32)
    outT = _gather_kernel(idx, embeddings.T)
    return outT.T


# speedup vs baseline: 3.5091x; 3.5091x over previous
import functools

import jax
import jax.numpy as jnp
from jax import lax
from jax.experimental import pallas as pl
from jax.experimental.pallas import tpu as pltpu
from jax.experimental.pallas import tpu_sc as plsc

VOCAB = 1000000
EMBED = 64
BATCH = 16384

NUM_CORES = 2
NUM_SUBCORES = 16
NUM_WORKERS = NUM_CORES * NUM_SUBCORES
LANES = 128
N_BLOCKS = (VOCAB + LANES - 1) // LANES
VOCAB_PAD = N_BLOCKS * LANES
CB = 4
CHUNK = CB * LANES
N_ROUNDS = 62
WL_CAP = BATCH + 16
ROW_SLOTS = 16
SENTINEL = 2**31 - 1

_mesh = plsc.VectorSubcoreMesh(core_axis_name="c", subcore_axis_name="s")


@functools.partial(
    pl.kernel,
    mesh=_mesh,
    out_type=(
        jax.ShapeDtypeStruct((BATCH, EMBED), jnp.float32),
        jax.ShapeDtypeStruct((ROW_SLOTS, EMBED), jnp.float32),
    ),
    scratch_types=[
        pltpu.VMEM((BATCH,), jnp.int32),
        pltpu.VMEM((WL_CAP,), jnp.int32),
        pltpu.VMEM((WL_CAP,), jnp.int32),
        pltpu.VMEM((2, EMBED, CHUNK), jnp.float32),
        pltpu.VMEM((ROW_SLOTS, EMBED), jnp.float32),
        pltpu.SemaphoreType.DMA,
        pltpu.SemaphoreType.DMA,
    ]
    + [pltpu.SemaphoreType.DMA] * ROW_SLOTS,
    compiler_params=pltpu.CompilerParams(
        use_tc_tiling_on_sc=True, needs_layout_passes=False
    ),
)
def _gather_kernel(
    idx_hbm,
    tableT_hbm,
    out_hbm,
    dummy_hbm,
    idx_v,
    wl_i,
    wl_p,
    chunks,
    rowstage,
    *sems,
):
    chunk_sems = sems[:2]
    row_sems = sems[2:]
    wid = lax.axis_index("s") * NUM_CORES + lax.axis_index("c")
    lo_block = lax.div(wid * N_BLOCKS, NUM_WORKERS)
    lo_lane = lo_block * LANES
    hi_lane = lax.div((wid + 1) * N_BLOCKS, NUM_WORKERS) * LANES

    pltpu.sync_copy(idx_hbm, idx_v)

    lane16 = lax.iota(jnp.int32, 16)
    row_vecs = [lane16 + (16 * j) for j in range(EMBED // 16)]

    def scan(v, count):
        iv = idx_v[pl.ds(v * 16, 16)]
        pv = lane16 + (v * 16)
        m = jnp.logical_and(iv >= lo_lane, iv < hi_lane)
        cnt = count
        plsc.store_compressed(wl_i.at[pl.ds(cnt, 16)], iv, mask=m)
        plsc.store_compressed(wl_p.at[pl.ds(cnt, 16)], pv, mask=m)
        return count + plsc.all_reduce_population_count(m)[0]

    count = lax.fori_loop(0, BATCH // 16, scan, jnp.int32(0))
    wl_i[pl.ds(count, 16)] = jnp.full((16,), SENTINEL, jnp.int32)
    n_wl_vecs = lax.div(count + 15, 16)

    for k in range(ROW_SLOTS):
        pltpu.async_copy(
            rowstage.at[pl.ds(k, 1)], dummy_hbm.at[pl.ds(k, 1)], row_sems[k]
        )

    def chunk_off(c):
        off = (lo_block + c * CB) * LANES
        return pl.multiple_of(jnp.minimum(off, VOCAB_PAD - CHUNK), LANES)

    def fire_chunk(buf, c):
        pltpu.async_copy(
            tableT_hbm.at[:, pl.ds(chunk_off(c), CHUNK)],
            chunks.at[buf],
            chunk_sems[buf],
        )

    def wait_chunk(buf):
        pltpu.make_async_copy(
            tableT_hbm.at[:, pl.ds(0, CHUNK)],
            chunks.at[buf],
            chunk_sems[buf],
        ).wait()

    def process(buf, c):
        c_lo = chunk_off(c)
        c_hi = c_lo + CHUNK

        def per_vec(w, carry):
            wv = wl_i[pl.ds(w * 16, 16)]
            m = jnp.logical_and(wv >= c_lo, wv < c_hi)
            npop = plsc.all_reduce_population_count(m)[0]

            @pl.when(npop > 0)
            def _():
                wp = wl_p[pl.ds(w * 16, 16)]
                mi = jnp.where(m, jnp.int32(1), jnp.int32(0))
                for k in range(16):
                    @pl.when(mi[k] == 1)
                    def _(k=k):
                        r = wv[k]
                        p = wp[k]
                        cc = jnp.full((16,), r - c_lo, jnp.int32)
                        pltpu.make_async_copy(
                            rowstage.at[pl.ds(k, 1)],
                            dummy_hbm.at[pl.ds(k, 1)],
                            row_sems[k],
                        ).wait()
                        for j, rows in enumerate(row_vecs):
                            vals = plsc.load_gather(chunks.at[buf], [rows, cc])
                            rowstage[k, pl.ds(16 * j, 16)] = vals
                        pltpu.async_copy(
                            rowstage.at[pl.ds(k, 1)],
                            out_hbm.at[pl.ds(p, 1)],
                            row_sems[k],
                        )

            return carry

        lax.fori_loop(0, n_wl_vecs, per_vec, 0)

    fire_chunk(0, 0)

    def dstep(d, carry):
        fire_chunk(1, 2 * d + 1)
        wait_chunk(0)
        process(0, 2 * d)
        fire_chunk(0, 2 * d + 2)
        wait_chunk(1)
        process(1, 2 * d + 1)
        return carry

    lax.fori_loop(0, N_ROUNDS // 2, dstep, 0)
    wait_chunk(0)

    for k in range(ROW_SLOTS):
        pltpu.make_async_copy(
            rowstage.at[pl.ds(k, 1)], dummy_hbm.at[pl.ds(k, 1)], row_sems[k]
        ).wait()


def kernel(inputs, embeddings):
    idx = inputs.astype(jnp.int32)
    out, _ = _gather_kernel(idx, embeddings.T)
    return out


# speedup vs baseline: 5.9611x; 1.6987x over previous
import functools

import jax
import jax.numpy as jnp
from jax import lax
from jax.experimental import pallas as pl
from jax.experimental.pallas import tpu as pltpu
from jax.experimental.pallas import tpu_sc as plsc

VOCAB = 1000000
EMBED = 64
BATCH = 16384

NUM_CORES = 2
NUM_SUBCORES = 16
NUM_WORKERS = NUM_CORES * NUM_SUBCORES
B_PER_WORKER = BATCH // NUM_WORKERS
LANES = 128
RING = 8
STEP = 16
N_STEPS = B_PER_WORKER // STEP

_mesh = plsc.VectorSubcoreMesh(core_axis_name="c", subcore_axis_name="s")


@functools.partial(
    pl.kernel,
    mesh=_mesh,
    out_type=jax.ShapeDtypeStruct((EMBED, BATCH), jnp.float32),
    scratch_types=[
        pltpu.VMEM((B_PER_WORKER,), jnp.int32),
        pltpu.VMEM((RING, EMBED, LANES), jnp.float32),
        pltpu.VMEM((EMBED, B_PER_WORKER), jnp.float32),
    ]
    + [pltpu.SemaphoreType.DMA] * RING,
    compiler_params=pltpu.CompilerParams(
        use_tc_tiling_on_sc=True, needs_layout_passes=False
    ),
)
def _gather_kernel(idx_hbm, tableT_hbm, outT_hbm, idx_v, blocks, colsT_v, *sems):
    wid = lax.axis_index("s") * NUM_CORES + lax.axis_index("c")
    base = wid * B_PER_WORKER
    pltpu.sync_copy(idx_hbm.at[pl.ds(base, B_PER_WORKER)], idx_v)

    row_vecs = [lax.iota(jnp.int32, 16) + (16 * j) for j in range(EMBED // 16)]

    def fire(b, r):
        blk = pl.multiple_of(lax.div(r, LANES) * LANES, LANES)
        pltpu.async_copy(
            tableT_hbm.at[:, pl.ds(blk, LANES)],
            blocks.at[b],
            sems[b],
        )

    def wait_extract(b, r, pos):
        pltpu.make_async_copy(
            tableT_hbm.at[:, pl.ds(0, LANES)],
            blocks.at[b],
            sems[b],
        ).wait()
        lane = jnp.bitwise_and(r, LANES - 1)
        col_ids = jnp.full((16,), lane, jnp.int32)
        pos_ids = jnp.full((16,), pos, jnp.int32)
        for rows in row_vecs:
            vals = plsc.load_gather(blocks.at[b], [rows, col_ids])
            plsc.store_scatter(colsT_v, [rows, pos_ids], vals)

    def step(s, vec_old):
        vec_cur = idx_v[pl.ds(s * STEP, STEP)]
        for k in range(STEP):
            b = k % RING
            if k < RING:
                @pl.when(s > 0)
                def _(k=k, b=b):
                    wait_extract(b, vec_old[k + RING], (s - 1) * STEP + k + RING)
            else:
                wait_extract(b, vec_cur[k - RING], s * STEP + k - RING)
            fire(b, vec_cur[k])
        return vec_cur

    lax.fori_loop(0, N_STEPS, step, idx_v[pl.ds(0, STEP)])

    vec_tail = idx_v[pl.ds(B_PER_WORKER - STEP, STEP)]
    for k in range(RING):
        wait_extract(k, vec_tail[k + RING], B_PER_WORKER - RING + k)

    pltpu.sync_copy(colsT_v, outT_hbm.at[:, pl.ds(base, B_PER_WORKER)])


def kernel(inputs, embeddings):
    idx = inputs.astype(jnp.int32)
    outT = _gather_kernel(idx, embeddings.T)
    return outT.T


# speedup vs baseline: 6.6478x; 1.1152x over previous
import functools

import jax
import jax.numpy as jnp
from jax import lax
from jax.experimental import pallas as pl
from jax.experimental.pallas import tpu as pltpu
from jax.experimental.pallas import tpu_sc as plsc

VOCAB = 1000000
EMBED = 64
BATCH = 16384

NUM_CORES = 2
NUM_SUBCORES = 16
NUM_WORKERS = NUM_CORES * NUM_SUBCORES
LANES = 128
N_BLOCKS = (VOCAB + LANES - 1) // LANES
VOCAB_PAD = N_BLOCKS * LANES
CB = 2
CHUNK = CB * LANES
N_ROUNDS = 124
N_BUCKETS = 128
WL_CAP = BATCH + 16 * N_ROUNDS + 16
ROW_SLOTS = 16
SENTINEL = 2**31 - 1

_mesh = plsc.VectorSubcoreMesh(core_axis_name="c", subcore_axis_name="s")


@functools.partial(
    pl.kernel,
    mesh=_mesh,
    out_type=(
        jax.ShapeDtypeStruct((BATCH, EMBED), jnp.float32),
        jax.ShapeDtypeStruct((ROW_SLOTS, EMBED), jnp.float32),
    ),
    scratch_types=[
        pltpu.VMEM((BATCH,), jnp.int32),
        pltpu.VMEM((WL_CAP,), jnp.int32),
        pltpu.VMEM((WL_CAP,), jnp.int32),
        pltpu.VMEM((WL_CAP,), jnp.int32),
        pltpu.VMEM((WL_CAP,), jnp.int32),
        pltpu.VMEM((N_BUCKETS,), jnp.int32),
        pltpu.VMEM((N_BUCKETS,), jnp.int32),
        pltpu.VMEM((N_BUCKETS,), jnp.int32),
        pltpu.VMEM((2, EMBED, CHUNK), jnp.float32),
        pltpu.VMEM((ROW_SLOTS, EMBED), jnp.float32),
        pltpu.SemaphoreType.DMA,
        pltpu.SemaphoreType.DMA,
    ]
    + [pltpu.SemaphoreType.DMA] * ROW_SLOTS,
    compiler_params=pltpu.CompilerParams(
        use_tc_tiling_on_sc=True, needs_layout_passes=False
    ),
)
def _gather_kernel(
    idx_hbm,
    tableT_hbm,
    out_hbm,
    dummy_hbm,
    idx_v,
    wl_i,
    wl_p,
    ws_i,
    ws_p,
    off_v,
    cur_v,
    nv_v,
    chunks,
    rowstage,
    *sems,
):
    chunk_sems = sems[:2]
    row_sems = sems[2:]
    wid = lax.axis_index("s") * NUM_CORES + lax.axis_index("c")
    lo_block = lax.div(wid * N_BLOCKS, NUM_WORKERS)
    lo_lane = lo_block * LANES
    hi_lane = lax.div((wid + 1) * N_BLOCKS, NUM_WORKERS) * LANES

    pltpu.sync_copy(idx_hbm, idx_v)

    lane16 = lax.iota(jnp.int32, 16)
    lane0 = lane16 == 0
    row_vecs = [lane16 + (16 * j) for j in range(EMBED // 16)]

    def bcast(x):
        return jnp.full((16,), x, jnp.int32)

    def scan(v, count):
        iv = idx_v[pl.ds(v * 16, 16)]
        pv = lane16 + (v * 16)
        m = jnp.logical_and(iv >= lo_lane, iv < hi_lane)
        plsc.store_compressed(wl_i.at[pl.ds(count, 16)], iv, mask=m)
        plsc.store_compressed(wl_p.at[pl.ds(count, 16)], pv, mask=m)
        return count + plsc.all_reduce_population_count(m)[0]

    count = lax.fori_loop(0, BATCH // 16, scan, jnp.int32(0))
    n_wl_vecs = lax.div(count + 15, 16)

    for g in range(N_BUCKETS // 16):
        cur_v[pl.ds(g * 16, 16)] = jnp.zeros((16,), jnp.int32)

    def bucket_of(r):
        return jnp.clip((r - lo_lane) // CHUNK, 0, N_ROUNDS - 1)

    def count_pass(w, carry):
        wv = wl_i[pl.ds(w * 16, 16)]
        valid = (lane16 + w * 16) < count
        vi = jnp.where(valid, jnp.int32(1), jnp.int32(0))
        bv = bucket_of(wv)
        for k in range(16):
            @pl.when(vi[k] == 1)
            def _(k=k):
                b = bv[k]
                c = plsc.load_gather(cur_v, [bcast(b)])
                plsc.store_scatter(cur_v, [bcast(b)], c + 1, mask=lane0)
        return carry

    lax.fori_loop(0, n_wl_vecs, count_pass, 0)

    def prefix(g, carry):
        c16 = cur_v[pl.ds(g * 16, 16)]
        pad16 = jnp.bitwise_and(c16 + 15, ~15)
        incl = plsc.cumsum(pad16)
        excl = incl - pad16 + carry
        off_v[pl.ds(g * 16, 16)] = excl
        nv_v[pl.ds(g * 16, 16)] = lax.div(pad16, 16)
        return excl[15] + pad16[15]

    lax.fori_loop(0, N_BUCKETS // 16, prefix, jnp.int32(0))
    for g in range(N_BUCKETS // 16):
        cur_v[pl.ds(g * 16, 16)] = off_v[pl.ds(g * 16, 16)]

    def fill(v, carry):
        ws_i[pl.ds(v * 16, 16)] = bcast(SENTINEL)
        return carry

    lax.fori_loop(0, WL_CAP // 16, fill, 0)

    def place(w, carry):
        wv = wl_i[pl.ds(w * 16, 16)]
        pv = wl_p[pl.ds(w * 16, 16)]
        valid = (lane16 + w * 16) < count
        vi = jnp.where(valid, jnp.int32(1), jnp.int32(0))
        bv = bucket_of(wv)
        for k in range(16):
            @pl.when(vi[k] == 1)
            def _(k=k):
                b = bv[k]
                slot = plsc.load_gather(cur_v, [bcast(b)])
                plsc.store_scatter(ws_i, [slot], bcast(wv[k]), mask=lane0)
                plsc.store_scatter(ws_p, [slot], bcast(pv[k]), mask=lane0)
                plsc.store_scatter(cur_v, [bcast(b)], slot + 1, mask=lane0)
        return carry

    lax.fori_loop(0, n_wl_vecs, place, 0)

    for k in range(ROW_SLOTS):
        pltpu.async_copy(
            rowstage.at[pl.ds(k, 1)], dummy_hbm.at[pl.ds(k, 1)], row_sems[k]
        )

    def chunk_off(c):
        off = (lo_block + c * CB) * LANES
        return pl.multiple_of(jnp.minimum(off, VOCAB_PAD - CHUNK), LANES)

    def fire_chunk(buf, c):
        pltpu.async_copy(
            tableT_hbm.at[:, pl.ds(chunk_off(c), CHUNK)],
            chunks.at[buf],
            chunk_sems[buf],
        )

    def wait_chunk(buf):
        pltpu.make_async_copy(
            tableT_hbm.at[:, pl.ds(0, CHUNK)],
            chunks.at[buf],
            chunk_sems[buf],
        ).wait()

    def process(buf, c):
        c_lo = chunk_off(c)
        start = plsc.load_gather(off_v, [bcast(c)])[0]
        nv = plsc.load_gather(nv_v, [bcast(c)])[0]

        def per_vec(w, carry):
            base = start + w * 16
            wv = ws_i[pl.ds(base, 16)]
            wp = ws_p[pl.ds(base, 16)]
            mi = jnp.where(wv != SENTINEL, jnp.int32(1), jnp.int32(0))
            for k in range(16):
                @pl.when(mi[k] == 1)
                def _(k=k):
                    r = wv[k]
                    p = wp[k]
                    cc = bcast(r - c_lo)
                    pltpu.make_async_copy(
                        rowstage.at[pl.ds(k, 1)],
                        dummy_hbm.at[pl.ds(k, 1)],
                        row_sems[k],
                    ).wait()
                    for j, rows in enumerate(row_vecs):
                        vals = plsc.load_gather(chunks.at[buf], [rows, cc])
                        rowstage[k, pl.ds(16 * j, 16)] = vals
                    pltpu.async_copy(
                        rowstage.at[pl.ds(k, 1)],
                        out_hbm.at[pl.ds(p, 1)],
                        row_sems[k],
                    )
            return carry

        lax.fori_loop(0, nv, per_vec, 0)

    fire_chunk(0, 0)

    def dstep(d, carry):
        fire_chunk(1, 2 * d + 1)
        wait_chunk(0)
        process(0, 2 * d)
        fire_chunk(0, 2 * d + 2)
        wait_chunk(1)
        process(1, 2 * d + 1)
        return carry

    lax.fori_loop(0, N_ROUNDS // 2, dstep, 0)
    wait_chunk(0)

    for k in range(ROW_SLOTS):
        pltpu.make_async_copy(
            rowstage.at[pl.ds(k, 1)], dummy_hbm.at[pl.ds(k, 1)], row_sems[k]
        ).wait()


def kernel(inputs, embeddings):
    idx = inputs.astype(jnp.int32)
    out, _ = _gather_kernel(idx, embeddings.T)
    return out


# speedup vs baseline: 6.8500x; 1.0304x over previous
import functools

import jax
import jax.numpy as jnp
from jax import lax
from jax.experimental import pallas as pl
from jax.experimental.pallas import tpu as pltpu
from jax.experimental.pallas import tpu_sc as plsc

VOCAB = 1000000
EMBED = 64
BATCH = 16384

NUM_CORES = 2
NUM_SUBCORES = 16
NUM_WORKERS = NUM_CORES * NUM_SUBCORES
LANES = 128
N_BLOCKS = (VOCAB + LANES - 1) // LANES
VOCAB_PAD = N_BLOCKS * LANES
CB = 2
CHUNK = CB * LANES
N_ROUNDS = 124
N_BUCKETS = 128
WL_CAP = BATCH + 16 * N_ROUNDS + 16
ROW_SLOTS = 16
SENTINEL = 2**31 - 1

_mesh = plsc.VectorSubcoreMesh(core_axis_name="c", subcore_axis_name="s")


@functools.partial(
    pl.kernel,
    mesh=_mesh,
    out_type=(
        jax.ShapeDtypeStruct((BATCH, EMBED), jnp.float32),
        jax.ShapeDtypeStruct((ROW_SLOTS, EMBED), jnp.float32),
    ),
    scratch_types=[
        pltpu.VMEM((BATCH,), jnp.int32),
        pltpu.VMEM((WL_CAP,), jnp.int32),
        pltpu.VMEM((WL_CAP,), jnp.int32),
        pltpu.VMEM((WL_CAP,), jnp.int32),
        pltpu.VMEM((WL_CAP,), jnp.int32),
        pltpu.VMEM((N_BUCKETS,), jnp.int32),
        pltpu.VMEM((N_BUCKETS,), jnp.int32),
        pltpu.VMEM((N_BUCKETS,), jnp.int32),
        pltpu.VMEM((2, EMBED, CHUNK), jnp.float32),
        pltpu.VMEM((ROW_SLOTS, EMBED), jnp.float32),
        pltpu.SemaphoreType.DMA,
        pltpu.SemaphoreType.DMA,
    ]
    + [pltpu.SemaphoreType.DMA] * ROW_SLOTS,
    compiler_params=pltpu.CompilerParams(
        use_tc_tiling_on_sc=True, needs_layout_passes=False
    ),
)
def _gather_kernel(
    idx_hbm,
    tableT_hbm,
    out_hbm,
    dummy_hbm,
    idx_v,
    wl_i,
    wl_p,
    ws_i,
    ws_p,
    off_v,
    cur_v,
    nv_v,
    chunks,
    rowstage,
    *sems,
):
    chunk_sems = sems[:2]
    row_sems = sems[2:]
    wid = lax.axis_index("s") * NUM_CORES + lax.axis_index("c")
    lo_block = lax.div(wid * N_BLOCKS, NUM_WORKERS)
    lo_lane = lo_block * LANES
    hi_lane = lax.div((wid + 1) * N_BLOCKS, NUM_WORKERS) * LANES

    def chunk_off(c):
        off = (lo_block + c * CB) * LANES
        return pl.multiple_of(jnp.minimum(off, VOCAB_PAD - CHUNK), LANES)

    def fire_chunk(buf, c):
        pltpu.async_copy(
            tableT_hbm.at[:, pl.ds(chunk_off(c), CHUNK)],
            chunks.at[buf],
            chunk_sems[buf],
        )

    def wait_chunk(buf):
        pltpu.make_async_copy(
            tableT_hbm.at[:, pl.ds(0, CHUNK)],
            chunks.at[buf],
            chunk_sems[buf],
        ).wait()

    fire_chunk(0, 0)

    pltpu.sync_copy(idx_hbm, idx_v)

    lane16 = lax.iota(jnp.int32, 16)
    lane0 = lane16 == 0
    row_vecs = [lane16 + (16 * j) for j in range(EMBED // 16)]

    def bcast(x):
        return jnp.full((16,), x, jnp.int32)

    def scan(v, count):
        iv = idx_v[pl.ds(v * 16, 16)]
        pv = lane16 + (v * 16)
        m = jnp.logical_and(iv >= lo_lane, iv < hi_lane)
        plsc.store_compressed(wl_i.at[pl.ds(count, 16)], iv, mask=m)
        plsc.store_compressed(wl_p.at[pl.ds(count, 16)], pv, mask=m)
        return count + plsc.all_reduce_population_count(m)[0]

    count = lax.fori_loop(0, BATCH // 16, scan, jnp.int32(0))
    n_wl_vecs = lax.div(count + 15, 16)

    for g in range(N_BUCKETS // 16):
        cur_v[pl.ds(g * 16, 16)] = jnp.zeros((16,), jnp.int32)

    def bucket_of(r):
        return jnp.clip((r - lo_lane) // CHUNK, 0, N_ROUNDS - 1)

    def count_pass(w, carry):
        wv = wl_i[pl.ds(w * 16, 16)]
        valid = (lane16 + w * 16) < count
        vi = jnp.where(valid, jnp.int32(1), jnp.int32(0))
        bv = bucket_of(wv)
        for k in range(16):
            @pl.when(vi[k] == 1)
            def _(k=k):
                b = bv[k]
                c = plsc.load_gather(cur_v, [bcast(b)])
                plsc.store_scatter(cur_v, [bcast(b)], c + 1, mask=lane0)
        return carry

    lax.fori_loop(0, n_wl_vecs, count_pass, 0)

    def prefix(g, carry):
        c16 = cur_v[pl.ds(g * 16, 16)]
        pad16 = jnp.bitwise_and(c16 + 15, ~15)
        incl = plsc.cumsum(pad16)
        excl = incl - pad16 + carry
        off_v[pl.ds(g * 16, 16)] = excl
        nv_v[pl.ds(g * 16, 16)] = lax.div(pad16, 16)
        return excl[15] + pad16[15]

    total = lax.fori_loop(0, N_BUCKETS // 16, prefix, jnp.int32(0))
    for g in range(N_BUCKETS // 16):
        cur_v[pl.ds(g * 16, 16)] = off_v[pl.ds(g * 16, 16)]

    def fill(v, carry):
        ws_i[pl.ds(v * 16, 16)] = bcast(SENTINEL)
        return carry

    lax.fori_loop(0, lax.div(total, 16), fill, 0)

    def place(w, carry):
        wv = wl_i[pl.ds(w * 16, 16)]
        pv = wl_p[pl.ds(w * 16, 16)]
        valid = (lane16 + w * 16) < count
        vi = jnp.where(valid, jnp.int32(1), jnp.int32(0))
        bv = bucket_of(wv)
        for k in range(16):
            @pl.when(vi[k] == 1)
            def _(k=k):
                b = bv[k]
                slot = plsc.load_gather(cur_v, [bcast(b)])
                plsc.store_scatter(ws_i, [slot], bcast(wv[k]), mask=lane0)
                plsc.store_scatter(ws_p, [slot], bcast(pv[k]), mask=lane0)
                plsc.store_scatter(cur_v, [bcast(b)], slot + 1, mask=lane0)
        return carry

    lax.fori_loop(0, n_wl_vecs, place, 0)

    for k in range(ROW_SLOTS):
        pltpu.async_copy(
            rowstage.at[pl.ds(k, 1)], dummy_hbm.at[pl.ds(k, 1)], row_sems[k]
        )

    def process(buf, c):
        c_lo = chunk_off(c)
        start = plsc.load_gather(off_v, [bcast(c)])[0]
        nv = plsc.load_gather(nv_v, [bcast(c)])[0]

        def per_vec(w, carry):
            base = start + w * 16
            wv = ws_i[pl.ds(base, 16)]
            wp = ws_p[pl.ds(base, 16)]
            mi = jnp.where(wv != SENTINEL, jnp.int32(1), jnp.int32(0))
            for k in range(16):
                @pl.when(mi[k] == 1)
                def _(k=k):
                    r = wv[k]
                    p = wp[k]
                    cc = bcast(r - c_lo)
                    pltpu.make_async_copy(
                        rowstage.at[pl.ds(k, 1)],
                        dummy_hbm.at[pl.ds(k, 1)],
                        row_sems[k],
                    ).wait()
                    for j, rows in enumerate(row_vecs):
                        vals = plsc.load_gather(chunks.at[buf], [rows, cc])
                        rowstage[k, pl.ds(16 * j, 16)] = vals
                    pltpu.async_copy(
                        rowstage.at[pl.ds(k, 1)],
                        out_hbm.at[pl.ds(p, 1)],
                        row_sems[k],
                    )
            return carry

        lax.fori_loop(0, nv, per_vec, 0)

    def dstep(d, carry):
        fire_chunk(1, 2 * d + 1)
        wait_chunk(0)
        process(0, 2 * d)
        fire_chunk(0, 2 * d + 2)
        wait_chunk(1)
        process(1, 2 * d + 1)
        return carry

    lax.fori_loop(0, N_ROUNDS // 2, dstep, 0)
    wait_chunk(0)

    for k in range(ROW_SLOTS):
        pltpu.make_async_copy(
            rowstage.at[pl.ds(k, 1)], dummy_hbm.at[pl.ds(k, 1)], row_sems[k]
        ).wait()


def kernel(inputs, embeddings):
    idx = inputs.astype(jnp.int32)
    out, _ = _gather_kernel(idx, embeddings.T)
    return out
